# baseline (speedup 1.0000x reference)
import functools

import jax
import jax.numpy as jnp
from jax import lax
from jax.experimental import pallas as pl
from jax.experimental.pallas import tpu as pltpu
from jax.experimental.pallas import tpu_sc as plsc

_LANES = 16
_BATCH = 64


def _make_lookup(B: int, D: int):
    info = plsc.get_sparse_core_info()
    NW = info.num_cores * info.num_subcores
    assert B % (8 * NW) == 0 and D % _LANES == 0
    b_per_w = B // NW
    n_chunks = D // _LANES
    n_groups = b_per_w // _LANES
    n_batches = (b_per_w + _BATCH - 1) // _BATCH + 1
    mesh = plsc.VectorSubcoreMesh(core_axis_name="c", subcore_axis_name="s")

    @functools.partial(
        pl.kernel,
        mesh=mesh,
        out_type=jax.ShapeDtypeStruct((B, D), jnp.float32),
        scratch_types=[
            pltpu.VMEM((b_per_w,), jnp.int32),
            pltpu.VMEM((2, D), jnp.float32),
            pltpu.VMEM((_BATCH, D), jnp.float32),
            pltpu.VMEM((_BATCH, D), jnp.float32),
            pltpu.VMEM((2 * n_batches * _BATCH,), jnp.int32),
            pltpu.VMEM((n_batches, _BATCH), jnp.int32),
            pltpu.VMEM((n_batches, _BATCH), jnp.int32),
            pltpu.VMEM((1, D), jnp.float32),
            pltpu.SemaphoreType.DMA,
            pltpu.SemaphoreType.DMA,
        ],
    )
    def lookup(
        idx_hbm,
        table_hbm,
        out_hbm,
        idx_v,
        tab_v,
        rep0_v,
        rep1_v,
        pl_v,
        pl0m_v,
        pl1m_v,
        row_v,
        isem,
        wsem,
    ):
        sid = lax.axis_index("s")
        wid = sid * info.num_cores + lax.axis_index("c")
        base = wid * b_per_w

        icopy = pltpu.make_async_copy(
            idx_hbm.at[pl.ds(base, b_per_w)], idx_v, isem
        )
        icopy.start()
        pltpu.sync_copy(table_hbm, tab_v)

        r0 = [tab_v[0, pl.ds(c * _LANES, _LANES)] for c in range(n_chunks)]
        r1 = [tab_v[1, pl.ds(c * _LANES, _LANES)] for c in range(n_chunks)]

        def rep_body(i, carry):
            for c in range(n_chunks):
                sl = pl.ds(c * _LANES, _LANES)
                rep0_v[i, sl] = r0[c]
                rep1_v[i, sl] = r1[c]
            return carry

        lax.fori_loop(0, _BATCH, rep_body, 0)

        basev = jnp.full((_LANES,), base, dtype=jnp.int32)
        for j in range(2 * n_batches * _BATCH // _LANES):
            sl = pl.ds(j * _LANES, _LANES)
            pl_v[sl] = basev

        icopy.wait()

        LIST1 = n_batches * _BATCH

        def compact_body(g, carry):
            p0, p1 = carry
            iv = idx_v[pl.ds(g * _LANES, _LANES)]
            gbase = base + g * _LANES
            for l in range(_LANES):
                val = iv[l]
                off = p0 + val * (LIST1 + p1 - p0)
                pl_v[pl.ds(off, _LANES)] = jnp.full(
                    (_LANES,), gbase + l, dtype=jnp.int32
                )
                p0 = p0 + 1 - val
                p1 = p1 + val
            return p0, p1

        p0, p1 = lax.fori_loop(
            0, n_groups, compact_body, (jnp.int32(0), jnp.int32(0))
        )
        pl_v[pl.ds(p0, _LANES)] = basev
        pl_v[pl.ds(LIST1 + p1, _LANES)] = basev

        for b in range(n_batches):
            for j in range(_BATCH // _LANES):
                sl = pl.ds(j * _LANES, _LANES)
                fl = pl.ds(b * _BATCH + j * _LANES, _LANES)
                f2 = pl.ds(LIST1 + b * _BATCH + j * _LANES, _LANES)
                pl0m_v[b, sl] = pl_v[fl]
                pl1m_v[b, sl] = pl_v[f2]

        nb0 = (p0 + _BATCH - 1) // _BATCH
        nb1 = (p1 + _BATCH - 1) // _BATCH

        def scat0_body(b, carry):
            pltpu.make_async_copy(rep0_v, out_hbm.at[pl0m_v.at[b]], wsem).start()
            return carry

        def scat1_body(b, carry):
            pltpu.make_async_copy(rep1_v, out_hbm.at[pl1m_v.at[b]], wsem).start()
            return carry

        lax.fori_loop(0, nb0, scat0_body, 0)
        lax.fori_loop(0, nb1, scat1_body, 0)

        def drain_body(b, carry):
            pltpu.make_async_copy(rep0_v, out_hbm.at[pl0m_v.at[0]], wsem).wait()
            return carry

        lax.fori_loop(0, nb0 + nb1, drain_body, 0)

        iv0 = idx_v[pl.ds(0, _LANES)]
        pred0 = iv0[0] == 1
        for c in range(n_chunks):
            row_v[0, pl.ds(c * _LANES, _LANES)] = jnp.where(pred0, r1[c], r0[c])
        pltpu.sync_copy(row_v, out_hbm.at[pl.ds(base, 1)])

    return lookup


def kernel(indices, table):
    B = indices.shape[0]
    D = table.shape[1]
    return _make_lookup(B, D)(indices, table)


# speedup vs baseline: 1.2185x; 1.2185x over previous
import functools

import jax
import jax.numpy as jnp
from jax import lax
from jax.experimental import pallas as pl
from jax.experimental.pallas import tpu as pltpu
from jax.experimental.pallas import tpu_sc as plsc

_LANES = 16


def _make_lookup(B: int, D: int):
    info = plsc.get_sparse_core_info()
    NS = info.num_subcores
    NW = info.num_cores * NS
    assert B % (8 * NW) == 0
    b_per_w = B // NW
    mesh = plsc.VectorSubcoreMesh(core_axis_name="c", subcore_axis_name="s")

    @functools.partial(
        pl.kernel,
        mesh=mesh,
        out_type=jax.ShapeDtypeStruct((B, D), jnp.float32),
        scratch_types=[
            pltpu.VMEM((b_per_w,), jnp.int32),
            pltpu.VMEM((b_per_w, D), jnp.float32),
            pltpu.VMEM_SHARED((2 * NS, D), jnp.float32),
            pltpu.SemaphoreType.DMA,
            pltpu.SemaphoreType.DMA,
            pltpu.SemaphoreType.DMA,
        ],
    )
    def lookup(
        idx_hbm, table_hbm, out_hbm, idx_v, rows_v, shared_tab, isem, gsem, wsem
    ):
        sid = lax.axis_index("s")
        wid = sid * info.num_cores + lax.axis_index("c")
        base = wid * b_per_w

        icopy = pltpu.make_async_copy(
            idx_hbm.at[pl.ds(base, b_per_w)], idx_v, isem
        )
        icopy.start()
        tcopy = pltpu.make_async_copy(
            table_hbm, shared_tab.at[pl.ds(2 * sid, 2)], gsem
        )
        tcopy.start()
        icopy.wait()

        off = jnp.full((_LANES,), 2 * sid, dtype=jnp.int32)
        for j in range(b_per_w // _LANES):
            sl = pl.ds(j * _LANES, _LANES)
            idx_v[sl] = idx_v[sl] + off
        tcopy.wait()

        if b_per_w == 512:
            bounds = [0, 64, 192, 320, 448, b_per_w]
        else:
            q = b_per_w // 4
            bounds = [0, q, 2 * q, 3 * q, b_per_w]
        pieces = list(zip(bounds[:-1], bounds[1:]))
        gathers = [
            pltpu.make_async_copy(
                shared_tab.at[idx_v.at[pl.ds(lo, hi - lo)]],
                rows_v.at[pl.ds(lo, hi - lo)],
                gsem,
            )
            for lo, hi in pieces
        ]
        writes = [
            pltpu.make_async_copy(
                rows_v.at[pl.ds(lo, hi - lo)],
                out_hbm.at[pl.ds(base + lo, hi - lo)],
                wsem,
            )
            for lo, hi in pieces
        ]
        gathers[0].start()
        for k in range(len(pieces)):
            if k + 1 < len(pieces):
                gathers[k + 1].start()
            gathers[k].wait()
            writes[k].start()
        for k in range(len(pieces)):
            writes[k].wait()

    return lookup


def kernel(indices, table):
    B = indices.shape[0]
    D = table.shape[1]
    return _make_lookup(B, D)(indices, table)
